# Initial kernel scaffold; baseline (speedup 1.0000x reference)
import functools

import jax
import jax.numpy as jnp
from jax import lax
from jax.experimental import pallas as pl
from jax.experimental.pallas import tpu as pltpu
from jax.experimental.pallas import tpu_sc as plsc

N = 100000
E = 3200000
D = 16

NC = 2
NS = 16
NW = NC * NS
G = 80
NROW = E // G
ROWS_PW = NROW // NW
K = 25
CHUNKS = ROWS_PW // K
RPS = N // NS

_mesh = plsc.VectorSubcoreMesh(core_axis_name="c", subcore_axis_name="s")


def _bcast_lane(v16, i):
    idx = jnp.full((16, 1), i, dtype=jnp.int32)
    dnums = lax.GatherDimensionNumbers(
        offset_dims=(), collapsed_slice_dims=(0,), start_index_map=(0,))
    return lax.gather(v16, idx, dnums, slice_sizes=(1,),
                      mode=lax.GatherScatterMode.PROMISE_IN_BOUNDS)


@functools.partial(
    pl.kernel,
    out_type=jax.ShapeDtypeStruct((2 * N, D), jnp.float32),
    mesh=_mesh,
    scratch_types=[
        pltpu.VMEM((K, G), jnp.int32),
        pltpu.VMEM((K, G), jnp.int32),
        pltpu.VMEM((K, G), jnp.float32),
        pltpu.VMEM((G, D), jnp.float32),
        pltpu.VMEM_SHARED((N, D), jnp.float32),
    ],
)
def _spmm_sc(src_hbm, dst_hbm, val_hbm, e_hbm, out_hbm,
             src_v, dst_v, val_v, rows_v, acc):
    cid = lax.axis_index("c")
    sid = lax.axis_index("s")
    wid = cid * NS + sid

    pltpu.sync_copy(e_hbm.at[pl.ds(sid * RPS, RPS)],
                    acc.at[pl.ds(sid * RPS, RPS)])
    plsc.subcore_barrier()

    base_row = wid * ROWS_PW

    @pl.loop(0, CHUNKS)
    def _chunk(t):
        row0 = base_row + t * K
        pltpu.sync_copy(src_hbm.at[pl.ds(row0, K)], src_v)
        pltpu.sync_copy(dst_hbm.at[pl.ds(row0, K)], dst_v)
        pltpu.sync_copy(val_hbm.at[pl.ds(row0, K)], val_v)

        @pl.loop(0, K)
        def _group(j):
            pltpu.sync_copy(e_hbm.at[src_v.at[j]], rows_v)
            for q in range(G // 16):
                vals16 = val_v[j, pl.ds(q * 16, 16)]
                for i in range(16):
                    r = q * 16 + i
                    rows_v[r, :] = rows_v[r, :] * _bcast_lane(vals16, i)
            pltpu.sync_copy(rows_v, acc.at[dst_v.at[j]], add=True)

    plsc.subcore_barrier()
    pltpu.sync_copy(acc.at[pl.ds(sid * RPS, RPS)],
                    out_hbm.at[pl.ds(cid * N + sid * RPS, RPS)])


def _combine_body_layer(p0_ref, p1_ref, e_ref, o_ref):
    o_ref[...] = p0_ref[...] + p1_ref[...] - e_ref[...]


def _combine_body_mean(e0_ref, q0_ref, q1_ref, o_ref):
    o_ref[...] = (e0_ref[...] + q0_ref[...] + q1_ref[...]) * (1.0 / 3.0)


def _combine(body, a, b, c):
    rows = (N * D) // 128
    blk = rows // 5
    a2, b2, c2 = (x.reshape(rows, 128) for x in (a, b, c))
    spec = pl.BlockSpec((blk, 128), lambda i: (i, 0))
    out = pl.pallas_call(
        body,
        out_shape=jax.ShapeDtypeStruct((rows, 128), jnp.float32),
        grid=(5,),
        in_specs=[spec, spec, spec],
        out_specs=spec,
    )(a2, b2, c2)
    return out.reshape(N, D)


def kernel(edge_index, edge_values, embedding_weight):
    dst2d = edge_index[0].astype(jnp.int32).reshape(NROW, G)
    src2d = edge_index[1].astype(jnp.int32).reshape(NROW, G)
    val2d = edge_values.reshape(NROW, G)
    e0 = embedding_weight

    p = _spmm_sc(src2d, dst2d, val2d, e0)
    e1 = _combine(_combine_body_layer, p[:N], p[N:], e0)
    q = _spmm_sc(src2d, dst2d, val2d, e1)
    return _combine(_combine_body_mean, e0, q[:N], q[N:])


# speedup vs baseline: 15.9214x; 15.9214x over previous
import functools

import jax
import jax.numpy as jnp
from jax import lax
from jax.experimental import pallas as pl
from jax.experimental.pallas import tpu as pltpu
from jax.experimental.pallas import tpu_sc as plsc

N = 100000
E = 3200000
D = 16

NC = 2
NS = 16
NW = NC * NS
G = 100
ROWS_PW = E // (G * NW)
K = 40
CHUNKS = ROWS_PW // K
SA = 6256
SA_LAST = N - (NS - 1) * SA

_mesh = plsc.VectorSubcoreMesh(core_axis_name="c", subcore_axis_name="s")


def _bcast_lane(v16, i):
    idx = jnp.full((16, 1), i, dtype=jnp.int32)
    dnums = lax.GatherDimensionNumbers(
        offset_dims=(), collapsed_slice_dims=(0,), start_index_map=(0,))
    return lax.gather(v16, idx, dnums, slice_sizes=(1,),
                      mode=lax.GatherScatterMode.PROMISE_IN_BOUNDS)


def _scale_rows(rows_v, val_v, j):
    for q in range(G // 16):
        vals16 = val_v[j, pl.ds(q * 16, 16)]
        for i in range(16):
            r = q * 16 + i
            rows_v[r, :] = rows_v[r, :] * _bcast_lane(vals16, i)
    rem = G % 16
    if rem:
        base = G - 16
        vals16 = val_v[j, pl.ds(base, 16)]
        for i in range(16 - rem, 16):
            r = base + i
            rows_v[r, :] = rows_v[r, :] * _bcast_lane(vals16, i)


@functools.partial(
    pl.kernel,
    out_type=jax.ShapeDtypeStruct((2 * N, D), jnp.float32),
    mesh=_mesh,
    compiler_params=pltpu.CompilerParams(use_tc_tiling_on_sc=False),
    scratch_types=[
        pltpu.VMEM((K, G), jnp.int32),
        pltpu.VMEM((K, G), jnp.int32),
        pltpu.VMEM((K, G), jnp.float32),
        pltpu.VMEM((G, D), jnp.float32),
        pltpu.VMEM_SHARED((N, D), jnp.float32),
    ],
)
def _spmm_sc(src_hbm, dst_hbm, val_hbm, e_hbm, out_hbm,
             src_v, dst_v, val_v, rows_v, acc):
    cid = lax.axis_index("c")
    sid = lax.axis_index("s")
    wid = cid * NS + sid

    stripe = pl.multiple_of(sid * SA, 8)

    @pl.when(sid < NS - 1)
    def _():
        pltpu.sync_copy(e_hbm.at[pl.ds(stripe, SA)], acc.at[pl.ds(stripe, SA)])

    @pl.when(sid == NS - 1)
    def _():
        pltpu.sync_copy(e_hbm.at[pl.ds(stripe, SA_LAST)],
                        acc.at[pl.ds(stripe, SA_LAST)])

    plsc.subcore_barrier()

    @pl.loop(0, CHUNKS)
    def _chunk(t):
        row0 = pl.multiple_of(t * K, 8)
        pltpu.sync_copy(src_hbm.at[wid, pl.ds(row0, K)], src_v)
        pltpu.sync_copy(dst_hbm.at[wid, pl.ds(row0, K)], dst_v)
        pltpu.sync_copy(val_hbm.at[wid, pl.ds(row0, K)], val_v)

        @pl.loop(0, K)
        def _group(j):
            pltpu.sync_copy(e_hbm.at[src_v.at[j]], rows_v)
            _scale_rows(rows_v, val_v, j)
            pltpu.sync_copy(rows_v, acc.at[dst_v.at[j]], add=True)

    plsc.subcore_barrier()
    out0 = pl.multiple_of(cid * N + sid * SA, 8)

    @pl.when(sid < NS - 1)
    def _():
        pltpu.sync_copy(acc.at[pl.ds(stripe, SA)], out_hbm.at[pl.ds(out0, SA)])

    @pl.when(sid == NS - 1)
    def _():
        pltpu.sync_copy(acc.at[pl.ds(stripe, SA_LAST)],
                        out_hbm.at[pl.ds(out0, SA_LAST)])


def _combine_body_layer(p0_ref, p1_ref, e_ref, o_ref):
    o_ref[...] = p0_ref[...] + p1_ref[...] - e_ref[...]


def _combine_body_mean(e0_ref, q0_ref, q1_ref, o_ref):
    o_ref[...] = (e0_ref[...] + q0_ref[...] + q1_ref[...]) * (1.0 / 3.0)


def _combine(body, a, b, c):
    rows = (N * D) // 128
    a2, b2, c2 = (x.reshape(rows, 128) for x in (a, b, c))
    out = pl.pallas_call(
        body,
        out_shape=jax.ShapeDtypeStruct((rows, 128), jnp.float32),
    )(a2, b2, c2)
    return out.reshape(N, D)


def kernel(edge_index, edge_values, embedding_weight):
    dst3d = edge_index[0].astype(jnp.int32).reshape(NW, ROWS_PW, G)
    src3d = edge_index[1].astype(jnp.int32).reshape(NW, ROWS_PW, G)
    val3d = edge_values.reshape(NW, ROWS_PW, G)
    e0 = embedding_weight

    p = _spmm_sc(src3d, dst3d, val3d, e0)
    e1 = _combine(_combine_body_layer, p[:N], p[N:], e0)
    q = _spmm_sc(src3d, dst3d, val3d, e1)
    return _combine(_combine_body_mean, e0, q[:N], q[N:])


# speedup vs baseline: 23.3129x; 1.4642x over previous
import functools

import jax
import jax.numpy as jnp
from jax import lax
from jax.experimental import pallas as pl
from jax.experimental.pallas import tpu as pltpu
from jax.experimental.pallas import tpu_sc as plsc

N = 100000
E = 3200000
D = 16

NC = 2
NS = 16
NW = NC * NS
G = 100
ROWS_PW = E // (G * NW)
K = 40
CHUNKS = ROWS_PW // K
SA = 6256
SA_LAST = N - (NS - 1) * SA

_mesh = plsc.VectorSubcoreMesh(core_axis_name="c", subcore_axis_name="s")


def _bcast_lane(v16, i):
    idx = jnp.full((16, 1), i, dtype=jnp.int32)
    dnums = lax.GatherDimensionNumbers(
        offset_dims=(), collapsed_slice_dims=(0,), start_index_map=(0,))
    return lax.gather(v16, idx, dnums, slice_sizes=(1,),
                      mode=lax.GatherScatterMode.PROMISE_IN_BOUNDS)


def _scale_rows(rows_v, val_v, j):
    for q in range(G // 16):
        vals16 = val_v[j, pl.ds(q * 16, 16)]
        for i in range(16):
            r = q * 16 + i
            rows_v[r, :] = rows_v[r, :] * _bcast_lane(vals16, i)
    rem = G % 16
    if rem:
        base = G - 16
        vals16 = val_v[j, pl.ds(base, 16)]
        for i in range(16 - rem, 16):
            r = base + i
            rows_v[r, :] = rows_v[r, :] * _bcast_lane(vals16, i)


@functools.partial(
    pl.kernel,
    out_type=jax.ShapeDtypeStruct((2 * N, D), jnp.float32),
    mesh=_mesh,
    compiler_params=pltpu.CompilerParams(use_tc_tiling_on_sc=False),
    scratch_types=[
        pltpu.VMEM((K, G), jnp.int32),
        pltpu.VMEM((K, G), jnp.int32),
        pltpu.VMEM((K, G), jnp.float32),
        pltpu.VMEM((G, D), jnp.float32),
        pltpu.VMEM((G, D), jnp.float32),
        pltpu.VMEM_SHARED((N, D), jnp.float32),
        pltpu.SemaphoreType.DMA,
        pltpu.SemaphoreType.DMA,
    ],
)
def _spmm_sc(src_hbm, dst_hbm, val_hbm, e_hbm, out_hbm,
             src_v, dst_v, val_v, rows0_v, rows1_v, acc, sem0, sem1):
    cid = lax.axis_index("c")
    sid = lax.axis_index("s")
    wid = cid * NS + sid

    stripe = pl.multiple_of(sid * SA, 8)

    @pl.when(sid < NS - 1)
    def _():
        pltpu.sync_copy(e_hbm.at[pl.ds(stripe, SA)], acc.at[pl.ds(stripe, SA)])

    @pl.when(sid == NS - 1)
    def _():
        pltpu.sync_copy(e_hbm.at[pl.ds(stripe, SA_LAST)],
                        acc.at[pl.ds(stripe, SA_LAST)])

    plsc.subcore_barrier()

    @pl.loop(0, CHUNKS)
    def _chunk(t):
        row0 = pl.multiple_of(t * K, 8)
        pltpu.sync_copy(src_hbm.at[wid, pl.ds(row0, K)], src_v)
        pltpu.sync_copy(dst_hbm.at[wid, pl.ds(row0, K)], dst_v)
        pltpu.sync_copy(val_hbm.at[wid, pl.ds(row0, K)], val_v)

        pltpu.async_copy(e_hbm.at[src_v.at[0]], rows0_v, sem0)

        @pl.loop(0, K, step=2)
        def _pair(j):
            bufs = ((rows0_v, sem0), (rows1_v, sem1))
            for b in range(2):
                rows_v, sem = bufs[b]
                nrows_v, nsem = bufs[1 - b]
                jj = j + b

                @pl.when(jj + 1 < K)
                def _():
                    pltpu.async_copy(e_hbm.at[src_v.at[jj + 1]], nrows_v, nsem)

                pltpu.make_async_copy(e_hbm.at[src_v.at[jj]], rows_v, sem).wait()
                _scale_rows(rows_v, val_v, jj)
                pltpu.sync_copy(rows_v, acc.at[dst_v.at[jj]], add=True)

    plsc.subcore_barrier()
    out0 = pl.multiple_of(cid * N + sid * SA, 8)

    @pl.when(sid < NS - 1)
    def _():
        pltpu.sync_copy(acc.at[pl.ds(stripe, SA)], out_hbm.at[pl.ds(out0, SA)])

    @pl.when(sid == NS - 1)
    def _():
        pltpu.sync_copy(acc.at[pl.ds(stripe, SA_LAST)],
                        out_hbm.at[pl.ds(out0, SA_LAST)])


def _combine_body_layer(p0_ref, p1_ref, e_ref, o_ref):
    o_ref[...] = p0_ref[...] + p1_ref[...] - e_ref[...]


def _combine_body_mean(e0_ref, q0_ref, q1_ref, o_ref):
    o_ref[...] = (e0_ref[...] + q0_ref[...] + q1_ref[...]) * (1.0 / 3.0)


def _combine(body, a, b, c):
    rows = (N * D) // 128
    a2, b2, c2 = (x.reshape(rows, 128) for x in (a, b, c))
    out = pl.pallas_call(
        body,
        out_shape=jax.ShapeDtypeStruct((rows, 128), jnp.float32),
    )(a2, b2, c2)
    return out.reshape(N, D)


def kernel(edge_index, edge_values, embedding_weight):
    dst3d = edge_index[0].astype(jnp.int32).reshape(NW, ROWS_PW, G)
    src3d = edge_index[1].astype(jnp.int32).reshape(NW, ROWS_PW, G)
    val3d = edge_values.reshape(NW, ROWS_PW, G)
    e0 = embedding_weight

    p = _spmm_sc(src3d, dst3d, val3d, e0)
    e1 = _combine(_combine_body_layer, p[:N], p[N:], e0)
    q = _spmm_sc(src3d, dst3d, val3d, e1)
    return _combine(_combine_body_mean, e0, q[:N], q[N:])
